# baseline (speedup 1.0000x reference)
import jax
import jax.numpy as jnp
from jax.experimental import pallas as pl
from jax.experimental.pallas import tpu as pltpu

N = 32768
D = 4096
E = 64
TAU = 1.0

TILE2 = 512


def _top2(logits):
    iota = jax.lax.broadcasted_iota(jnp.int32, logits.shape, 1)
    m1 = jnp.max(logits, axis=1, keepdims=True)
    idx1 = jnp.min(jnp.where(logits == m1, iota, E), axis=1, keepdims=True)
    masked = jnp.where(iota == idx1, -jnp.inf, logits)
    m2 = jnp.max(masked, axis=1, keepdims=True)
    idx2 = jnp.min(jnp.where(masked == m2, iota, E), axis=1, keepdims=True)
    w1 = 1.0 / (1.0 + jnp.exp(m2 - m1))
    p = jnp.exp(logits - m1)
    probs = p / jnp.sum(p, axis=1, keepdims=True)
    return idx1, idx2, w1, probs


def _router_kernel(x2_ref, w_ref, b_ref, aux_ref, probs_ref, logits_ref):
    w = w_ref[...]
    b = b_ref[...]
    inv_tau = 1.0 / max(TAU, 1e-06)
    halves = []
    for h in (0, 1):
        xh = x2_ref[:, pl.ds(h * D, D)]
        lg = jax.lax.dot_general(
            xh, w, (((1,), (1,)), ((), ())),
            preferred_element_type=jnp.float32,
        )
        lg = lg + b
        if inv_tau != 1.0:
            lg = lg * inv_tau
        halves.append(lg)

    le, lo = halves
    logits_ref[:, :E] = le
    logits_ref[:, E:] = lo

    i1e, i2e, w1e, pe = _top2(le)
    i1o, i2o, w1o, po = _top2(lo)
    probs_ref[:, :E] = pe
    probs_ref[:, E:] = po

    aux_ref[...] = jnp.concatenate(
        [i1e.astype(jnp.float32), i2e.astype(jnp.float32), w1e, 1.0 - w1e,
         i1o.astype(jnp.float32), i2o.astype(jnp.float32), w1o, 1.0 - w1o],
        axis=1,
    )


@jax.jit
def kernel(x, W, b):
    x2 = x.reshape(N // 2, 2 * D)
    b2 = b.reshape(1, E)
    grid = ((N // 2) // TILE2,)
    out_shapes = (
        jax.ShapeDtypeStruct((N // 2, 8), jnp.float32),
        jax.ShapeDtypeStruct((N // 2, 2 * E), jnp.float32),
        jax.ShapeDtypeStruct((N // 2, 2 * E), jnp.float32),
    )
    aux, probs_pair, logits_pair = pl.pallas_call(
        _router_kernel,
        grid=grid,
        in_specs=[
            pl.BlockSpec((TILE2, 2 * D), lambda i: (i, 0)),
            pl.BlockSpec((E, D), lambda i: (0, 0)),
            pl.BlockSpec((1, E), lambda i: (0, 0)),
        ],
        out_specs=(
            pl.BlockSpec((TILE2, 8), lambda i: (i, 0)),
            pl.BlockSpec((TILE2, 2 * E), lambda i: (i, 0)),
            pl.BlockSpec((TILE2, 2 * E), lambda i: (i, 0)),
        ),
        out_shape=out_shapes,
        compiler_params=pltpu.CompilerParams(
            dimension_semantics=("arbitrary",),
        ),
    )(x2, W, b2)

    aux4 = aux.reshape(N, 4)
    top_idx = aux4[:, :2].astype(jnp.int32)
    top_w = aux4[:, 2:4]
    probs_full = probs_pair.reshape(N, E)
    logits = logits_pair.reshape(N, E)
    return (top_idx, top_w, probs_full, logits)


# speedup vs baseline: 1.0247x; 1.0247x over previous
import jax
import jax.numpy as jnp
from jax.experimental import pallas as pl
from jax.experimental.pallas import tpu as pltpu

N = 32768
D = 4096
E = 64
TAU = 1.0

TILE2 = 512


def _top2(logits):
    iota = jax.lax.broadcasted_iota(jnp.int32, logits.shape, 1)
    m1 = jnp.max(logits, axis=1, keepdims=True)
    idx1 = jnp.min(jnp.where(logits == m1, iota, E), axis=1, keepdims=True)
    masked = jnp.where(iota == idx1, -jnp.inf, logits)
    m2 = jnp.max(masked, axis=1, keepdims=True)
    idx2 = jnp.min(jnp.where(masked == m2, iota, E), axis=1, keepdims=True)
    w1 = 1.0 / (1.0 + jnp.exp(m2 - m1))
    p = jnp.exp(logits - m1)
    probs = p / jnp.sum(p, axis=1, keepdims=True)
    return idx1, idx2, w1, probs


def _router_kernel(xe_ref, xo_ref, w_ref, b_ref, aux_ref, probs_ref, logits_ref):
    w = w_ref[...]
    b = b_ref[...]
    inv_tau = 1.0 / max(TAU, 1e-06)
    halves = []
    for xh_ref in (xe_ref, xo_ref):
        lg = jax.lax.dot_general(
            xh_ref[...], w, (((1,), (1,)), ((), ())),
            preferred_element_type=jnp.float32,
        )
        lg = lg + b
        if inv_tau != 1.0:
            lg = lg * inv_tau
        halves.append(lg)

    le, lo = halves
    logits_ref[:, :E] = le
    logits_ref[:, E:] = lo

    i1e, i2e, w1e, pe = _top2(le)
    i1o, i2o, w1o, po = _top2(lo)
    probs_ref[:, :E] = pe
    probs_ref[:, E:] = po

    aux_ref[...] = jnp.concatenate(
        [i1e.astype(jnp.float32), i2e.astype(jnp.float32), w1e, 1.0 - w1e,
         i1o.astype(jnp.float32), i2o.astype(jnp.float32), w1o, 1.0 - w1o],
        axis=1,
    )


@jax.jit
def kernel(x, W, b):
    x2 = x.reshape(N // 2, 2 * D)
    b2 = b.reshape(1, E)
    grid = ((N // 2) // TILE2,)
    out_shapes = (
        jax.ShapeDtypeStruct((N // 2, 8), jnp.float32),
        jax.ShapeDtypeStruct((N // 2, 2 * E), jnp.float32),
        jax.ShapeDtypeStruct((N // 2, 2 * E), jnp.float32),
    )
    aux, probs_pair, logits_pair = pl.pallas_call(
        _router_kernel,
        grid=grid,
        in_specs=[
            pl.BlockSpec((TILE2, D), lambda i: (i, 0)),
            pl.BlockSpec((TILE2, D), lambda i: (i, 1)),
            pl.BlockSpec((E, D), lambda i: (0, 0)),
            pl.BlockSpec((1, E), lambda i: (0, 0)),
        ],
        out_specs=(
            pl.BlockSpec((TILE2, 8), lambda i: (i, 0)),
            pl.BlockSpec((TILE2, 2 * E), lambda i: (i, 0)),
            pl.BlockSpec((TILE2, 2 * E), lambda i: (i, 0)),
        ),
        out_shape=out_shapes,
        compiler_params=pltpu.CompilerParams(
            dimension_semantics=("arbitrary",),
        ),
    )(x2, x2, W, b2)

    aux4 = aux.reshape(N, 4)
    top_idx = aux4[:, :2].astype(jnp.int32)
    top_w = aux4[:, 2:4]
    probs_full = probs_pair.reshape(N, E)
    logits = logits_pair.reshape(N, E)
    return (top_idx, top_w, probs_full, logits)


# speedup vs baseline: 3.0938x; 3.0193x over previous
import jax
import jax.numpy as jnp
from jax.experimental import pallas as pl
from jax.experimental.pallas import tpu as pltpu

N = 32768
D = 4096
E = 64
TAU = 1.0

TILE = 1024


def _router_kernel(x_ref, w_ref, b_ref, out_ref):
    logits = jax.lax.dot_general(
        x_ref[...], w_ref[...], (((1,), (1,)), ((), ())),
        preferred_element_type=jnp.float32,
    )
    logits = logits + b_ref[...]
    inv_tau = 1.0 / max(TAU, 1e-06)
    if inv_tau != 1.0:
        logits = logits * inv_tau
    out_ref[:, :E] = logits

    iota = jax.lax.broadcasted_iota(jnp.int32, logits.shape, 1)
    m1 = jnp.max(logits, axis=1, keepdims=True)
    idx1 = jnp.min(jnp.where(logits == m1, iota, E), axis=1, keepdims=True)
    masked = jnp.where(iota == idx1, -jnp.inf, logits)
    m2 = jnp.max(masked, axis=1, keepdims=True)
    idx2 = jnp.min(jnp.where(masked == m2, iota, E), axis=1, keepdims=True)

    p = jnp.exp(logits - m1)
    out_ref[:, E:2 * E] = p / jnp.sum(p, axis=1, keepdims=True)

    w1 = 1.0 / (1.0 + jnp.exp(m2 - m1))
    out_ref[:, 2 * E:] = jnp.concatenate(
        [idx1.astype(jnp.float32), idx2.astype(jnp.float32), w1, 1.0 - w1],
        axis=1,
    )


@jax.jit
def kernel(x, W, b):
    b2 = b.reshape(1, E)
    grid = (N // TILE,)
    combined = pl.pallas_call(
        _router_kernel,
        grid=grid,
        in_specs=[
            pl.BlockSpec((TILE, D), lambda i: (i, 0)),
            pl.BlockSpec((E, D), lambda i: (0, 0)),
            pl.BlockSpec((1, E), lambda i: (0, 0)),
        ],
        out_specs=pl.BlockSpec((TILE, 2 * E + 4), lambda i: (i, 0)),
        out_shape=jax.ShapeDtypeStruct((N, 2 * E + 4), jnp.float32),
        compiler_params=pltpu.CompilerParams(
            dimension_semantics=("arbitrary",),
        ),
    )(x, W, b2)

    logits = combined[:, :E]
    probs_full = combined[:, E:2 * E]
    top_idx = combined[:, 2 * E:2 * E + 2].astype(jnp.int32)
    top_w = combined[:, 2 * E + 2:2 * E + 4]
    return (top_idx, top_w, probs_full, logits)


# speedup vs baseline: 4.2480x; 1.3731x over previous
import jax
import jax.numpy as jnp
from jax.experimental import pallas as pl
from jax.experimental.pallas import tpu as pltpu

N = 32768
D = 4096
E = 64
TAU = 1.0

TILE = 1024


def _router_kernel(x_ref, w_in_ref, b_ref, aux_ref, probs_ref, logits_ref):
    logits = jax.lax.dot_general(
        x_ref[...], w_in_ref[...], (((1,), (1,)), ((), ())),
        preferred_element_type=jnp.float32,
    )
    logits = logits + b_ref[...]
    inv_tau = 1.0 / max(TAU, 1e-06)
    if inv_tau != 1.0:
        logits = logits * inv_tau
    logits_ref[...] = logits

    iota = jax.lax.broadcasted_iota(jnp.int32, logits.shape, 1)
    m1 = jnp.max(logits, axis=1, keepdims=True)
    idx1 = jnp.min(jnp.where(logits == m1, iota, E), axis=1, keepdims=True)
    masked = jnp.where(iota == idx1, -jnp.inf, logits)
    m2 = jnp.max(masked, axis=1, keepdims=True)
    idx2 = jnp.min(jnp.where(masked == m2, iota, E), axis=1, keepdims=True)

    w1 = 1.0 / (1.0 + jnp.exp(m2 - m1))
    aux = jnp.concatenate(
        [idx1.astype(jnp.float32), idx2.astype(jnp.float32), w1, 1.0 - w1],
        axis=1,
    )
    aux_ref[...] = aux.T

    p = jnp.exp(logits - m1)
    probs_ref[...] = p / jnp.sum(p, axis=1, keepdims=True)


@jax.jit
def kernel(x, W, b):
    b2 = b.reshape(1, E)
    grid = (N // TILE,)
    out_shapes = (
        jax.ShapeDtypeStruct((4, N), jnp.float32),
        jax.ShapeDtypeStruct((N, E), jnp.float32),
        jax.ShapeDtypeStruct((N, E), jnp.float32),
    )
    row_specE = pl.BlockSpec((TILE, E), lambda i: (i, 0))
    aux_t, probs_full, logits = pl.pallas_call(
        _router_kernel,
        grid=grid,
        in_specs=[
            pl.BlockSpec((TILE, D), lambda i: (i, 0)),
            pl.BlockSpec((E, D), lambda i: (0, 0)),
            pl.BlockSpec((1, E), lambda i: (0, 0)),
        ],
        out_specs=(
            pl.BlockSpec((4, TILE), lambda i: (0, i)),
            row_specE,
            row_specE,
        ),
        out_shape=out_shapes,
        compiler_params=pltpu.CompilerParams(
            dimension_semantics=("arbitrary",),
        ),
    )(x, W, b2)
    top_idx = aux_t[:2].T.astype(jnp.int32)
    top_w = aux_t[2:].T
    return (top_idx, top_w, probs_full, logits)
